# baseline (speedup 1.0000x reference)
import functools

import jax
import jax.numpy as jnp
from jax import lax
from jax.experimental import pallas as pl
from jax.experimental.pallas import tpu as pltpu
from jax.experimental.pallas import tpu_sc as plsc

NC, NS, LANES = 2, 16, 16
NW = NC * NS

_N = 10000
_E = 320000
_D = 128
_H = 64

EPW = _E // NW
CH = 40
NCHUNK = EPW // CH
AW = _D
RPW = 624
TAIL = _N - NS * RPW

def _leaky(v):
    return jnp.maximum(v, 0.3 * v)


CH1 = 128
NCHT = _E // CH1
BASE1 = NCHT // NW
EXTRA1 = NCHT - BASE1 * NW
IMAX1 = BASE1 + 1


def _sc_rel_body(src_hbm, dst_hbm, pos4_hbm, relt_hbm, pos_v, isrc, idst,
                 relbuf, semr0, semr1):
    c = lax.axis_index("c")
    s = lax.axis_index("s")
    wid = s * NC + c
    semr = (semr0, semr1)
    sw = BASE1 * wid + jnp.minimum(wid, EXTRA1)
    cnt = BASE1 + (wid < EXTRA1).astype(jnp.int32)
    e0 = sw * CH1
    pltpu.sync_copy(pos4_hbm, pos_v)
    pltpu.sync_copy(src_hbm.at[pl.ds(e0, BASE1 * CH1)],
                    isrc.at[pl.ds(0, BASE1 * CH1)])
    pltpu.sync_copy(dst_hbm.at[pl.ds(e0, BASE1 * CH1)],
                    idst.at[pl.ds(0, BASE1 * CH1)])

    @pl.when(wid < EXTRA1)
    def _():
        pltpu.sync_copy(src_hbm.at[pl.ds(e0 + BASE1 * CH1, CH1)],
                        isrc.at[pl.ds(BASE1 * CH1, CH1)])
        pltpu.sync_copy(dst_hbm.at[pl.ds(e0 + BASE1 * CH1, CH1)],
                        idst.at[pl.ds(BASE1 * CH1, CH1)])

    def build(i, b):
        for g in range(CH1 // LANES):
            sv = isrc[pl.ds(i * CH1 + g * LANES, LANES)] * 4
            dv = idst[pl.ds(i * CH1 + g * LANES, LANES)] * 4
            for cc in range(4):
                ps = plsc.load_gather(pos_v, [sv + cc])
                pd = plsc.load_gather(pos_v, [dv + cc])
                relbuf[b, cc, pl.ds(g * LANES, LANES)] = ps - pd

    def out_dma(i, b):
        pltpu.async_copy(relbuf.at[b],
                         relt_hbm.at[:, pl.ds((sw + i) * CH1, CH1)], semr[b])

    def out_wait(i, b):
        pltpu.make_async_copy(relbuf.at[b],
                              relt_hbm.at[:, pl.ds((sw + i) * CH1, CH1)],
                              semr[b]).wait()

    def chunk(i, carry):
        for b in range(2):
            i2 = i * 2 + b

            @pl.when(i2 < cnt)
            def _():
                @pl.when(i2 >= 2)
                def _():
                    out_wait(i2 - 2, b)

                build(i2, b)
                out_dma(i2, b)

        return carry

    lax.fori_loop(0, (IMAX1 + 1) // 2, chunk, 0)

    last0 = ((cnt - 1) // 2) * 2
    last1 = ((cnt - 2) // 2) * 2 + 1
    out_wait(last0, 0)
    out_wait(last1, 1)


@functools.cache
def _sc_kernels():
    mesh = plsc.VectorSubcoreMesh(core_axis_name="c", subcore_axis_name="s",
                                  num_cores=NC, num_subcores=NS)
    sc_rel = pl.kernel(
        _sc_rel_body,
        out_type=jax.ShapeDtypeStruct((4, _E), jnp.float32),
        mesh=mesh,
        scratch_types=[
            pltpu.VMEM((4 * _N,), jnp.float32),
            pltpu.VMEM((IMAX1 * CH1,), jnp.int32),
            pltpu.VMEM((IMAX1 * CH1,), jnp.int32),
            pltpu.VMEM((2, 4, CH1), jnp.float32),
            pltpu.SemaphoreType.DMA,
            pltpu.SemaphoreType.DMA,
        ],
        compiler_params=pltpu.CompilerParams(needs_layout_passes=False),
    )
    sc_agg = pl.kernel(
        _sc_agg_body,
        out_type=jax.ShapeDtypeStruct((NC, _N, AW), jnp.float32),
        mesh=mesh,
        scratch_types=[
            pltpu.VMEM((EPW,), jnp.int32),
            pltpu.VMEM((4, CH), jnp.int32),
            pltpu.VMEM((2, CH, _H), jnp.float32),
            pltpu.VMEM((2, CH, _D), jnp.float32),
            pltpu.VMEM((2, CH, AW), jnp.float32),
            pltpu.VMEM_SHARED((_N, AW), jnp.float32),
            pltpu.SemaphoreType.DMA,
            pltpu.SemaphoreType.DMA,
            pltpu.SemaphoreType.DMA,
            pltpu.SemaphoreType.DMA,
            pltpu.SemaphoreType.DMA,
            pltpu.SemaphoreType.DMA,
            pltpu.SemaphoreType.DMA,
            pltpu.SemaphoreType.DMA,
        ],
        compiler_params=pltpu.CompilerParams(needs_layout_passes=False),
    )
    return sc_rel, sc_agg


def _sc_agg_body(src_hbm, dst_hbm, w_hbm, res_hbm, z_hbm,
                 agg_out,
                 isrc_all, idring, wbuf, rbuf, mbuf, agg_sh,
                 semg0, semg1, semw0, semw1, semsc0, semsc1, semi0, semi1):
    c = lax.axis_index("c")
    s = lax.axis_index("s")
    wid = s * NC + c
    semg = (semg0, semg1)
    semw = (semw0, semw1)
    semsc = (semsc0, semsc1)
    semi = (semi0, semi1)

    def mkones(i, carry):
        for b in range(2):
            mbuf[b, i, pl.ds(_H, LANES)] = jnp.ones((LANES,), jnp.float32)
            for k in range(_H + LANES, AW, LANES):
                mbuf[b, i, pl.ds(k, LANES)] = jnp.zeros((LANES,), jnp.float32)
        return carry

    lax.fori_loop(0, CH, mkones, 0)
    pltpu.sync_copy(src_hbm.at[pl.ds(wid * EPW, EPW)], isrc_all)
    pltpu.sync_copy(z_hbm.at[pl.ds(0, RPW)], agg_sh.at[pl.ds(s * RPW, RPW)])

    @pl.when(s == NS - 1)
    def _():
        pltpu.sync_copy(z_hbm.at[pl.ds(0, TAIL)],
                        agg_sh.at[pl.ds(NS * RPW, TAIL)])

    plsc.subcore_barrier()

    def fetch_idx(i, r, q):
        pltpu.async_copy(dst_hbm.at[pl.ds(wid * EPW + i * CH, CH)],
                         idring.at[r], semi[q])

    def wait_idx(i, r, q):
        pltpu.make_async_copy(dst_hbm.at[pl.ds(wid * EPW + i * CH, CH)],
                              idring.at[r], semi[q]).wait()

    def issue(i, b):
        pltpu.async_copy(res_hbm.at[isrc_all.at[pl.ds(i * CH, CH)]],
                         rbuf.at[b], semg[b])
        pltpu.async_copy(w_hbm.at[pl.ds(wid * EPW + i * CH, CH)],
                         wbuf.at[b], semw[b])

    def wait_gw(i, b):
        pltpu.make_async_copy(res_hbm.at[isrc_all.at[pl.ds(i * CH, CH)]],
                              rbuf.at[b], semg[b]).wait()
        pltpu.make_async_copy(w_hbm.at[pl.ds(wid * EPW + i * CH, CH)],
                              wbuf.at[b], semw[b]).wait()

    def issue_sc(i, r, b):
        pltpu.async_copy(mbuf.at[b], agg_sh.at[idring.at[r]], semsc[b],
                         add=True)

    def wait_sc(i, r, b):
        pltpu.make_async_copy(mbuf.at[b], agg_sh.at[idring.at[r]],
                              semsc[b]).wait()

    fetch_idx(0, 0, 0)
    fetch_idx(1, 1, 1)
    issue(0, 0)
    npair = NCHUNK // 2

    def pair(p, carry):
        for b in range(2):
            i2 = p * 2 + b
            r = i2 % 4
            q = b
            wait_gw(i2, b)

            @pl.when(i2 + 1 < NCHUNK)
            def _():
                issue(i2 + 1, 1 - b)

            @pl.when(i2 >= 2)
            def _():
                wait_sc(i2 - 2, (i2 - 2) % 4, b)

            for rr in range(CH):
                for k in range(_H // LANES):
                    sl = pl.ds(k * LANES, LANES)
                    mbuf[b, rr, sl] = rbuf[b, rr, sl] * wbuf[b, rr, sl]
            wait_idx(i2, r, q)

            @pl.when(i2 + 2 < NCHUNK)
            def _():
                fetch_idx(i2 + 2, (i2 + 2) % 4, q)

            issue_sc(i2, r, b)

        return carry

    lax.fori_loop(0, npair, pair, 0)
    wait_sc(NCHUNK - 2, (NCHUNK - 2) % 4, 0)
    wait_sc(NCHUNK - 1, (NCHUNK - 1) % 4, 1)
    plsc.subcore_barrier()
    pltpu.sync_copy(agg_sh.at[pl.ds(s * RPW, RPW)],
                    agg_out.at[c, pl.ds(s * RPW, RPW)])

    @pl.when(s == NS - 1)
    def _():
        pltpu.sync_copy(agg_sh.at[pl.ds(NS * RPW, TAIL)],
                        agg_out.at[c, pl.ds(NS * RPW, TAIL)])


def _tc_res_body(x_ref, w1_ref, b1_ref, g1_ref, be1_ref, o_ref):
    o_ref[...] = _leaky(
        g1_ref[...] * (x_ref[...] @ w1_ref[...] + b1_ref[...]) + be1_ref[...])


def _tc_w_body(relt_ref, wm14_ref, bm1_ref, wm2_ref, o_ref):
    pre = lax.dot_general(relt_ref[...], wm14_ref[...],
                          (((0,), (0,)), ((), ())),
                          preferred_element_type=jnp.float32)
    h = _leaky(pre + bm1_ref[...])
    o_ref[...] = h @ wm2_ref[...]


def _tc_fin_body(x_ref, agg_ref, w2_ref, b2_ref, g2_ref, be2_ref,
                 g3_ref, be3_ref, ws_ref, bs_ref, g4_ref, be4_ref, o_ref):
    a = agg_ref[0] + agg_ref[1]
    agg = a[:, :_H]
    deg = a[:, _H:_H + 1]
    aggn = agg / jnp.maximum(deg, 1.0)
    r2 = _leaky(g2_ref[...] * aggn + be2_ref[...])
    r3 = g3_ref[...] * (r2 @ w2_ref[...] + b2_ref[...]) + be3_ref[...]
    sk = g4_ref[...] * (x_ref[...] @ ws_ref[...] + bs_ref[...]) + be4_ref[...]
    o_ref[...] = _leaky(r3 + sk)


_BN = 1000
_BE = 6400


def _full(shape):
    return pl.BlockSpec(shape, lambda i: (0,) * len(shape))


def _tc_res(x, W1p, b1p, g1p, be1p):
    return pl.pallas_call(
        _tc_res_body,
        grid=(_N // _BN,),
        in_specs=[
            pl.BlockSpec((_BN, _D), lambda i: (i, 0)),
            _full((_D, _D)), _full((_D,)), _full((_D,)), _full((_D,)),
        ],
        out_specs=pl.BlockSpec((_BN, _D), lambda i: (i, 0)),
        out_shape=jax.ShapeDtypeStruct((_N, _D), jnp.float32),
    )(x, W1p, b1p, g1p, be1p)


def _tc_w(relt, Wm1_4, bm1, Wm2):
    kh = Wm1_4.shape[1]
    return pl.pallas_call(
        _tc_w_body,
        grid=(_E // _BE,),
        in_specs=[
            pl.BlockSpec((4, _BE), lambda i: (0, i)),
            _full((4, kh)), _full((kh,)), _full((kh, _H)),
        ],
        out_specs=pl.BlockSpec((_BE, _H), lambda i: (i, 0)),
        out_shape=jax.ShapeDtypeStruct((_E, _H), jnp.float32),
    )(relt, Wm1_4, bm1, Wm2)


def _tc_fin(x, aggP, W2, b2, g2, be2, g3, be3, Ws, bs, g4, be4):
    return pl.pallas_call(
        _tc_fin_body,
        grid=(_N // _BN,),
        in_specs=[
            pl.BlockSpec((_BN, _D), lambda i: (i, 0)),
            pl.BlockSpec((NC, _BN, AW), lambda i: (0, i, 0)),
            _full((_H, _D)), _full((_D,)), _full((_H,)), _full((_H,)),
            _full((_D,)), _full((_D,)),
            _full((_D, _D)), _full((_D,)), _full((_D,)), _full((_D,)),
        ],
        out_specs=pl.BlockSpec((_BN, _D), lambda i: (i, 0)),
        out_shape=jax.ShapeDtypeStruct((_N, _D), jnp.float32),
    )(x, aggP, W2, b2, g2, be2, g3, be3, Ws, bs, g4, be4)


def kernel(x, pos, edge_index, W1, b1, g1, be1, Wm1, bm1, Wm2, g2, be2,
           W2, b2, g3, be3, Ws, bs, g4, be4):
    src = edge_index[0]
    dst = edge_index[1]
    pos4 = jnp.pad(pos, ((0, 0), (0, 1))).reshape(-1)
    Wm1_4 = jnp.concatenate([Wm1, jnp.zeros((1, Wm1.shape[1]), jnp.float32)],
                            axis=0)
    z = jnp.zeros((RPW, AW), jnp.float32)
    padH = jnp.zeros((_D - _H,), jnp.float32)
    W1p = jnp.concatenate([W1, jnp.zeros((_D, _D - _H), jnp.float32)], axis=1)
    b1p = jnp.concatenate([b1, padH])
    g1p = jnp.concatenate([g1, padH])
    be1p = jnp.concatenate([be1, padH])

    sc_rel, sc_agg = _sc_kernels()
    relt = sc_rel(src, dst, pos4)
    res = _tc_res(x, W1p, b1p, g1p, be1p)
    w = _tc_w(relt, Wm1_4, bm1, Wm2)
    aggP = sc_agg(src, dst, w, res, z)
    return _tc_fin(x, aggP, W2, b2, g2, be2, g3, be3, Ws, bs, g4, be4)


# speedup vs baseline: 1.0432x; 1.0432x over previous
import functools

import jax
import jax.numpy as jnp
from jax import lax
from jax.experimental import pallas as pl
from jax.experimental.pallas import tpu as pltpu
from jax.experimental.pallas import tpu_sc as plsc

NC, NS, LANES = 2, 16, 16
NW = NC * NS

_N = 10000
_E = 320000
_D = 128
_H = 64

EPW = _E // NW
CH = 40
NCHUNK = EPW // CH
AW = _D
RPW = 624
TAIL = _N - NS * RPW

def _leaky(v):
    return jnp.maximum(v, 0.3 * v)


CH1 = 128
NCHT = _E // CH1
BASE1 = NCHT // NW
EXTRA1 = NCHT - BASE1 * NW
IMAX1 = BASE1 + 1


def _sc_rel_body(src_hbm, dst_hbm, pos4_hbm, relt_hbm, pos_v, isrc, idst,
                 relbuf, semr0, semr1):
    c = lax.axis_index("c")
    s = lax.axis_index("s")
    wid = s * NC + c
    semr = (semr0, semr1)
    sw = BASE1 * wid + jnp.minimum(wid, EXTRA1)
    cnt = BASE1 + (wid < EXTRA1).astype(jnp.int32)
    e0 = sw * CH1
    pltpu.sync_copy(pos4_hbm, pos_v)
    pltpu.sync_copy(src_hbm.at[pl.ds(e0, BASE1 * CH1)],
                    isrc.at[pl.ds(0, BASE1 * CH1)])
    pltpu.sync_copy(dst_hbm.at[pl.ds(e0, BASE1 * CH1)],
                    idst.at[pl.ds(0, BASE1 * CH1)])

    @pl.when(wid < EXTRA1)
    def _():
        pltpu.sync_copy(src_hbm.at[pl.ds(e0 + BASE1 * CH1, CH1)],
                        isrc.at[pl.ds(BASE1 * CH1, CH1)])
        pltpu.sync_copy(dst_hbm.at[pl.ds(e0 + BASE1 * CH1, CH1)],
                        idst.at[pl.ds(BASE1 * CH1, CH1)])

    def build(i, b):
        for g in range(CH1 // LANES):
            sv = isrc[pl.ds(i * CH1 + g * LANES, LANES)] * 4
            dv = idst[pl.ds(i * CH1 + g * LANES, LANES)] * 4
            for cc in range(4):
                ps = plsc.load_gather(pos_v, [sv + cc])
                pd = plsc.load_gather(pos_v, [dv + cc])
                relbuf[b, cc, pl.ds(g * LANES, LANES)] = ps - pd

    def out_dma(i, b):
        pltpu.async_copy(relbuf.at[b],
                         relt_hbm.at[:, pl.ds((sw + i) * CH1, CH1)], semr[b])

    def out_wait(i, b):
        pltpu.make_async_copy(relbuf.at[b],
                              relt_hbm.at[:, pl.ds((sw + i) * CH1, CH1)],
                              semr[b]).wait()

    def chunk(i, carry):
        for b in range(2):
            i2 = i * 2 + b

            @pl.when(i2 < cnt)
            def _():
                @pl.when(i2 >= 2)
                def _():
                    out_wait(i2 - 2, b)

                build(i2, b)
                out_dma(i2, b)

        return carry

    lax.fori_loop(0, (IMAX1 + 1) // 2, chunk, 0)

    last0 = ((cnt - 1) // 2) * 2
    last1 = ((cnt - 2) // 2) * 2 + 1
    out_wait(last0, 0)
    out_wait(last1, 1)


@functools.cache
def _sc_kernels():
    mesh = plsc.VectorSubcoreMesh(core_axis_name="c", subcore_axis_name="s",
                                  num_cores=NC, num_subcores=NS)
    sc_rel = pl.kernel(
        _sc_rel_body,
        out_type=jax.ShapeDtypeStruct((4, _E), jnp.float32),
        mesh=mesh,
        scratch_types=[
            pltpu.VMEM((4 * _N,), jnp.float32),
            pltpu.VMEM((IMAX1 * CH1,), jnp.int32),
            pltpu.VMEM((IMAX1 * CH1,), jnp.int32),
            pltpu.VMEM((2, 4, CH1), jnp.float32),
            pltpu.SemaphoreType.DMA,
            pltpu.SemaphoreType.DMA,
        ],
        compiler_params=pltpu.CompilerParams(needs_layout_passes=False),
    )
    sc_agg = pl.kernel(
        _sc_agg_body,
        out_type=jax.ShapeDtypeStruct((NC, _N, AW), jnp.float32),
        mesh=mesh,
        scratch_types=[
            pltpu.VMEM((EPW,), jnp.int32),
            pltpu.VMEM((4, CH), jnp.int32),
            pltpu.VMEM((2, CH, _H), jnp.float32),
            pltpu.VMEM((2, CH, _D), jnp.float32),
            pltpu.VMEM((2, CH, AW), jnp.float32),
            pltpu.VMEM_SHARED((_N, AW), jnp.float32),
            pltpu.SemaphoreType.DMA,
            pltpu.SemaphoreType.DMA,
            pltpu.SemaphoreType.DMA,
            pltpu.SemaphoreType.DMA,
            pltpu.SemaphoreType.DMA,
            pltpu.SemaphoreType.DMA,
            pltpu.SemaphoreType.DMA,
            pltpu.SemaphoreType.DMA,
        ],
        compiler_params=pltpu.CompilerParams(needs_layout_passes=False),
    )
    return sc_rel, sc_agg


def _sc_agg_body(src_hbm, dst_hbm, w_hbm, res_hbm, z_hbm,
                 agg_out,
                 isrc_all, idring, wbuf, rbuf, mbuf, agg_sh,
                 semg0, semg1, semw0, semw1, semsc0, semsc1, semi0, semi1):
    c = lax.axis_index("c")
    s = lax.axis_index("s")
    wid = s * NC + c
    semg = (semg0, semg1)
    semw = (semw0, semw1)
    semsc = (semsc0, semsc1)
    semi = (semi0, semi1)

    def mkones(i, carry):
        for b in range(2):
            mbuf[b, i, pl.ds(_H, LANES)] = jnp.ones((LANES,), jnp.float32)
            for k in range(_H + LANES, AW, LANES):
                mbuf[b, i, pl.ds(k, LANES)] = jnp.zeros((LANES,), jnp.float32)
        return carry

    lax.fori_loop(0, CH, mkones, 0)
    pltpu.sync_copy(src_hbm.at[pl.ds(wid * EPW, EPW)], isrc_all)
    pltpu.sync_copy(z_hbm.at[pl.ds(0, RPW)], agg_sh.at[pl.ds(s * RPW, RPW)])

    @pl.when(s == NS - 1)
    def _():
        pltpu.sync_copy(z_hbm.at[pl.ds(0, TAIL)],
                        agg_sh.at[pl.ds(NS * RPW, TAIL)])

    plsc.subcore_barrier()

    def fetch_idx(i, r, q):
        pltpu.async_copy(dst_hbm.at[pl.ds(wid * EPW + i * CH, CH)],
                         idring.at[r], semi[q])

    def wait_idx(i, r, q):
        pltpu.make_async_copy(dst_hbm.at[pl.ds(wid * EPW + i * CH, CH)],
                              idring.at[r], semi[q]).wait()

    def issue(i, b):
        pltpu.async_copy(res_hbm.at[isrc_all.at[pl.ds(i * CH, CH)]],
                         rbuf.at[b], semg[b])
        pltpu.async_copy(w_hbm.at[pl.ds(wid * EPW + i * CH, CH)],
                         wbuf.at[b], semw[b])

    def wait_gw(i, b):
        pltpu.make_async_copy(res_hbm.at[isrc_all.at[pl.ds(i * CH, CH)]],
                              rbuf.at[b], semg[b]).wait()
        pltpu.make_async_copy(w_hbm.at[pl.ds(wid * EPW + i * CH, CH)],
                              wbuf.at[b], semw[b]).wait()

    def issue_sc(i, r, b):
        pltpu.async_copy(mbuf.at[b], agg_sh.at[idring.at[r]], semsc[b],
                         add=True)

    def wait_sc(i, r, b):
        pltpu.make_async_copy(mbuf.at[b], agg_sh.at[idring.at[r]],
                              semsc[b]).wait()

    fetch_idx(0, 0, 0)
    fetch_idx(1, 1, 1)
    issue(0, 0)
    npair = NCHUNK // 2

    def pair(p, carry):
        for b in range(2):
            i2 = p * 2 + b
            r = i2 % 4
            q = b
            wait_gw(i2, b)

            @pl.when(i2 + 1 < NCHUNK)
            def _():
                issue(i2 + 1, 1 - b)

            @pl.when(i2 >= 2)
            def _():
                wait_sc(i2 - 2, (i2 - 2) % 4, b)

            for rr in range(CH):
                for k in range(_H // LANES):
                    sl = pl.ds(k * LANES, LANES)
                    mbuf[b, rr, sl] = rbuf[b, rr, sl] * wbuf[b, rr, sl]
            wait_idx(i2, r, q)

            @pl.when(i2 + 2 < NCHUNK)
            def _():
                fetch_idx(i2 + 2, (i2 + 2) % 4, q)

            issue_sc(i2, r, b)

        return carry

    lax.fori_loop(0, npair, pair, 0)
    wait_sc(NCHUNK - 2, (NCHUNK - 2) % 4, 0)
    wait_sc(NCHUNK - 1, (NCHUNK - 1) % 4, 1)
    plsc.subcore_barrier()
    pltpu.sync_copy(agg_sh.at[pl.ds(s * RPW, RPW)],
                    agg_out.at[c, pl.ds(s * RPW, RPW)])

    @pl.when(s == NS - 1)
    def _():
        pltpu.sync_copy(agg_sh.at[pl.ds(NS * RPW, TAIL)],
                        agg_out.at[c, pl.ds(NS * RPW, TAIL)])


def _tc_rw_body(x_ref, w1_ref, b1_ref, g1_ref, be1_ref,
                relt_ref, wm14_ref, bm1_ref, wm2_ref, res_ref, w_ref):
    res_ref[...] = _leaky(
        g1_ref[...] * (x_ref[...] @ w1_ref[...] + b1_ref[...]) + be1_ref[...])
    pre = lax.dot_general(relt_ref[...], wm14_ref[...],
                          (((0,), (0,)), ((), ())),
                          preferred_element_type=jnp.float32)
    h = _leaky(pre + bm1_ref[...])
    w_ref[...] = h @ wm2_ref[...]


def _tc_fin_body(x_ref, agg_ref, w2_ref, b2_ref, g2_ref, be2_ref,
                 g3_ref, be3_ref, ws_ref, bs_ref, g4_ref, be4_ref, o_ref):
    a = agg_ref[0] + agg_ref[1]
    agg = a[:, :_H]
    deg = a[:, _H:_H + 1]
    aggn = agg / jnp.maximum(deg, 1.0)
    r2 = _leaky(g2_ref[...] * aggn + be2_ref[...])
    r3 = g3_ref[...] * (r2 @ w2_ref[...] + b2_ref[...]) + be3_ref[...]
    sk = g4_ref[...] * (x_ref[...] @ ws_ref[...] + bs_ref[...]) + be4_ref[...]
    o_ref[...] = _leaky(r3 + sk)


_BN = 1000
_BE = 6400


def _full(shape):
    return pl.BlockSpec(shape, lambda i: (0,) * len(shape))


def _tc_rw(x, W1p, b1p, g1p, be1p, relt, Wm1_4, bm1, Wm2):
    kh = Wm1_4.shape[1]
    eb = _E // (_N // _BN)
    return pl.pallas_call(
        _tc_rw_body,
        grid=(_N // _BN,),
        in_specs=[
            pl.BlockSpec((_BN, _D), lambda i: (i, 0)),
            _full((_D, _D)), _full((_D,)), _full((_D,)), _full((_D,)),
            pl.BlockSpec((4, eb), lambda i: (0, i)),
            _full((4, kh)), _full((kh,)), _full((kh, _H)),
        ],
        out_specs=[
            pl.BlockSpec((_BN, _D), lambda i: (i, 0)),
            pl.BlockSpec((eb, _H), lambda i: (i, 0)),
        ],
        out_shape=[
            jax.ShapeDtypeStruct((_N, _D), jnp.float32),
            jax.ShapeDtypeStruct((_E, _H), jnp.float32),
        ],
    )(x, W1p, b1p, g1p, be1p, relt, Wm1_4, bm1, Wm2)


def _tc_fin(x, aggP, W2, b2, g2, be2, g3, be3, Ws, bs, g4, be4):
    return pl.pallas_call(
        _tc_fin_body,
        grid=(_N // _BN,),
        in_specs=[
            pl.BlockSpec((_BN, _D), lambda i: (i, 0)),
            pl.BlockSpec((NC, _BN, AW), lambda i: (0, i, 0)),
            _full((_H, _D)), _full((_D,)), _full((_H,)), _full((_H,)),
            _full((_D,)), _full((_D,)),
            _full((_D, _D)), _full((_D,)), _full((_D,)), _full((_D,)),
        ],
        out_specs=pl.BlockSpec((_BN, _D), lambda i: (i, 0)),
        out_shape=jax.ShapeDtypeStruct((_N, _D), jnp.float32),
    )(x, aggP, W2, b2, g2, be2, g3, be3, Ws, bs, g4, be4)


def kernel(x, pos, edge_index, W1, b1, g1, be1, Wm1, bm1, Wm2, g2, be2,
           W2, b2, g3, be3, Ws, bs, g4, be4):
    src = edge_index[0]
    dst = edge_index[1]
    pos4 = jnp.pad(pos, ((0, 0), (0, 1))).reshape(-1)
    Wm1_4 = jnp.concatenate([Wm1, jnp.zeros((1, Wm1.shape[1]), jnp.float32)],
                            axis=0)
    z = jnp.zeros((RPW, AW), jnp.float32)
    padH = jnp.zeros((_D - _H,), jnp.float32)
    W1p = jnp.concatenate([W1, jnp.zeros((_D, _D - _H), jnp.float32)], axis=1)
    b1p = jnp.concatenate([b1, padH])
    g1p = jnp.concatenate([g1, padH])
    be1p = jnp.concatenate([be1, padH])

    sc_rel, sc_agg = _sc_kernels()
    relt = sc_rel(src, dst, pos4)
    res, w = _tc_rw(x, W1p, b1p, g1p, be1p, relt, Wm1_4, bm1, Wm2)
    aggP = sc_agg(src, dst, w, res, z)
    return _tc_fin(x, aggP, W2, b2, g2, be2, g3, be3, Ws, bs, g4, be4)
